# baseline (speedup 1.0000x reference)
import functools

import numpy as np
import jax
import jax.numpy as jnp
from jax import lax
from jax.experimental import pallas as pl
from jax.experimental.pallas import tpu as pltpu

DIM_INNER = 12
HDIM = DIM_INNER // 3
DPAD = 16
TN = 128
S = 32
CLASS_WIDTHS = (768, 896, 1024)


def _build_static_topology(seed=0, B=2048):
    rng = np.random.default_rng(seed)
    graph_sizes = rng.integers(112, 145, size=B).astype(np.int64)
    edge_lists = []
    for n_g in graph_sizes:
        m_g = int(3 * n_g)
        src = rng.integers(0, n_g, m_g)
        dst = rng.integers(0, n_g, m_g)
        edge_lists.append(np.stack([src, dst], axis=1))
    lg_node_idx = np.concatenate(edge_lists, axis=0).astype(np.int64)
    edge_counts = np.array([e.shape[0] for e in edge_lists], np.int64)
    ptr = np.concatenate([np.zeros(1, np.int64), np.cumsum(edge_counts)])
    return lg_node_idx, ptr, graph_sizes


def _precompute():
    lg_node_idx, ptr, graph_sizes = _build_static_topology()
    E = int(lg_node_idx.shape[0])
    node_off = np.concatenate([np.zeros(1, np.int64), np.cumsum(graph_sizes)])
    pad = np.repeat(node_off[:-1], (ptr[1:] - ptr[:-1]))
    idx0 = (lg_node_idx[:, 0] + pad).astype(np.int64)
    idx1 = (lg_node_idx[:, 1] + pad).astype(np.int64)
    n_out = int(max(idx0.max(), idx1.max())) + 1

    n_tiles = -(-n_out // TN)
    n_tiles_pad = -(-n_tiles // (2 * S)) * (2 * S)

    g_of_node_lo = np.searchsorted(node_off, np.arange(n_tiles) * TN, side="right") - 1
    g_of_node_hi = np.searchsorted(
        node_off, np.minimum(np.arange(n_tiles) * TN + TN - 1, n_out - 1),
        side="right") - 1
    estart = ptr[g_of_node_lo]
    eend = ptr[g_of_node_hi + 1]
    span = np.zeros(n_tiles_pad, np.int64)
    span[:n_tiles] = eend - (estart // TN) * TN
    WB = int(-(-span.max() // TN) * TN)
    E_pad = -(-max(E, WB) // TN) * TN + WB

    ws_all = np.zeros(n_tiles_pad, np.int64)
    ws_all[:n_tiles] = np.minimum((estart // TN) * TN, E_pad - WB)

    idx0_p = np.full(E_pad, -1, np.int64); idx0_p[:E] = idx0
    idx1_p = np.full(E_pad, -1, np.int64); idx1_p[:E] = idx1

    widths = sorted(set(w for w in CLASS_WIDTHS if w < WB)) + [WB]
    desc = np.argsort(-span, kind="stable")
    classes = []
    taken = 0
    for i in range(len(widths) - 1, -1, -1):
        lo = widths[i - 1] if i > 0 else -1
        if i > 0:
            need = int((span > lo).sum()) - taken
            n_w = max(0, -(-need // S) * S)
        else:
            n_w = n_tiles_pad - taken
        classes.append((widths[i], np.sort(desc[taken:taken + n_w])))
        taken += n_w
    classes = classes[::-1]

    n_pad = n_tiles_pad * TN
    cnt0 = np.bincount(idx0, minlength=n_pad).astype(np.float32)
    cnt1 = np.bincount(idx1, minlength=n_pad).astype(np.float32)
    r0 = (1.0 / np.maximum(cnt0, 1.0)).reshape(n_tiles_pad, TN)
    r1 = (1.0 / np.maximum(cnt1, 1.0)).reshape(n_tiles_pad, TN)

    class_data = []
    order_parts = []
    for w, tiles in classes:
        assert tiles.size % S == 0 and (span[tiles] <= w).all()
        ws = ws_all[tiles]
        gather = ws[:, None] + np.arange(w)[None, :]
        tbase = (tiles.astype(np.int64) * TN)[:, None]
        loc0 = (idx0_p[gather] - tbase).astype(np.int32)
        loc1 = (idx1_p[gather] - tbase).astype(np.int32)
        class_data.append(dict(W=w, n=tiles.size, ws=ws.astype(np.int32),
                               loc0=loc0, loc1=loc1,
                               r0=r0[tiles], r1=r1[tiles]))
        order_parts.append(tiles)
    order = np.concatenate(order_parts)
    assert order.size == n_tiles_pad
    inv_order = np.argsort(order).astype(np.int32)

    return dict(E=E, E_pad=E_pad, n_out=n_out, n_tiles_pad=n_tiles_pad,
                inv_order=inv_order, classes=class_data)


_P = _precompute()


def _body(W, ws_ref, loc0_ref, loc1_ref, r0_ref, r1_ref, xT_ref, o_ref):
    step = pl.program_id(0)
    for s in range(S):
        t = step * S + s
        start = pl.multiple_of(ws_ref[t], TN)
        xw = xT_ref[:, pl.ds(start, W)].astype(jnp.float32)
        iota = lax.broadcasted_iota(jnp.int32, (TN, W), 0)
        one = jnp.ones((), jnp.float32)
        zero = jnp.zeros((), jnp.float32)
        m0 = jnp.where(loc0_ref[s:s + 1, :] == iota, one, zero)
        m1 = jnp.where(loc1_ref[s:s + 1, :] == iota, one, zero)
        m = jnp.concatenate([m0, m1], axis=0)
        r = lax.dot_general(xw, m, (((1,), (1,)), ((), ())),
                            preferred_element_type=jnp.float32)
        o0 = r[:, :TN] * r0_ref[s:s + 1, :]
        o1 = r[:, TN:] * r1_ref[s:s + 1, :]
        rio = lax.broadcasted_iota(jnp.int32, (DPAD, TN), 0)
        mixed = jnp.where(rio < HDIM, (o1 - o0) * 0.5,
                          jnp.where(rio < 2 * HDIM, o1, o0))
        o_ref[:, s * TN:(s + 1) * TN] = mixed


def _scatter_call(W, n_class, ws, loc0, loc1, r0, r1, xT):
    steps = n_class // S
    return pl.pallas_call(
        functools.partial(_body, W),
        out_shape=jax.ShapeDtypeStruct((DPAD, n_class * TN), jnp.float32),
        grid_spec=pltpu.PrefetchScalarGridSpec(
            num_scalar_prefetch=1,
            grid=(steps,),
            in_specs=[
                pl.BlockSpec((S, W), lambda j, ws: (j, 0)),
                pl.BlockSpec((S, W), lambda j, ws: (j, 0)),
                pl.BlockSpec((S, TN), lambda j, ws: (j, 0)),
                pl.BlockSpec((S, TN), lambda j, ws: (j, 0)),
                pl.BlockSpec((DPAD, xT.shape[1]), lambda j, ws: (0, 0)),
            ],
            out_specs=pl.BlockSpec((DPAD, S * TN), lambda j, ws: (0, j)),
        ),
        compiler_params=pltpu.CompilerParams(
            dimension_semantics=("arbitrary",),
            vmem_limit_bytes=60 * 1024 * 1024),
        name=f"lg2graph_node_w{W}",
    )(jnp.asarray(ws), jnp.asarray(loc0), jnp.asarray(loc1),
      jnp.asarray(r0), jnp.asarray(r1), xT)


@jax.jit
def kernel(x):
    E_pad = _P["E_pad"]

    xT = jnp.pad(x.astype(jnp.bfloat16).T,
                 ((0, DPAD - DIM_INNER), (0, E_pad - _P["E"])))

    outs = [_scatter_call(c["W"], c["n"], c["ws"], c["loc0"], c["loc1"],
                          c["r0"], c["r1"], xT)
            for c in _P["classes"] if c["n"]]

    cat = jnp.concatenate(outs, axis=1) if len(outs) > 1 else outs[0]
    cat3 = cat.reshape(DPAD, _P["n_tiles_pad"], TN)
    full = jnp.take(cat3, jnp.asarray(_P["inv_order"]), axis=1)
    return full.reshape(DPAD, -1)[:DIM_INNER, :_P["n_out"]].T


# speedup vs baseline: 1.0172x; 1.0172x over previous
import functools

import numpy as np
import jax
import jax.numpy as jnp
from jax import lax
from jax.experimental import pallas as pl
from jax.experimental.pallas import tpu as pltpu

DIM_INNER = 12
HDIM = DIM_INNER // 3
DPAD = 16
TN = 128
S = 32
CLASS_WIDTHS = (896, 1024)


def _build_static_topology(seed=0, B=2048):
    rng = np.random.default_rng(seed)
    graph_sizes = rng.integers(112, 145, size=B).astype(np.int64)
    edge_lists = []
    for n_g in graph_sizes:
        m_g = int(3 * n_g)
        src = rng.integers(0, n_g, m_g)
        dst = rng.integers(0, n_g, m_g)
        edge_lists.append(np.stack([src, dst], axis=1))
    lg_node_idx = np.concatenate(edge_lists, axis=0).astype(np.int64)
    edge_counts = np.array([e.shape[0] for e in edge_lists], np.int64)
    ptr = np.concatenate([np.zeros(1, np.int64), np.cumsum(edge_counts)])
    return lg_node_idx, ptr, graph_sizes


def _precompute():
    lg_node_idx, ptr, graph_sizes = _build_static_topology()
    E = int(lg_node_idx.shape[0])
    node_off = np.concatenate([np.zeros(1, np.int64), np.cumsum(graph_sizes)])
    pad = np.repeat(node_off[:-1], (ptr[1:] - ptr[:-1]))
    idx0 = (lg_node_idx[:, 0] + pad).astype(np.int64)
    idx1 = (lg_node_idx[:, 1] + pad).astype(np.int64)
    n_out = int(max(idx0.max(), idx1.max())) + 1

    n_tiles = -(-n_out // TN)
    n_tiles_pad = -(-n_tiles // (2 * S)) * (2 * S)

    g_of_node_lo = np.searchsorted(node_off, np.arange(n_tiles) * TN, side="right") - 1
    g_of_node_hi = np.searchsorted(
        node_off, np.minimum(np.arange(n_tiles) * TN + TN - 1, n_out - 1),
        side="right") - 1
    estart = ptr[g_of_node_lo]
    eend = ptr[g_of_node_hi + 1]
    span = np.zeros(n_tiles_pad, np.int64)
    span[:n_tiles] = eend - (estart // TN) * TN
    WB = int(-(-span.max() // TN) * TN)
    E_pad = -(-max(E, WB) // TN) * TN + WB

    ws_all = np.zeros(n_tiles_pad, np.int64)
    ws_all[:n_tiles] = np.minimum((estart // TN) * TN, E_pad - WB)

    idx0_p = np.full(E_pad, -1, np.int64); idx0_p[:E] = idx0
    idx1_p = np.full(E_pad, -1, np.int64); idx1_p[:E] = idx1

    widths = sorted(set(w for w in CLASS_WIDTHS if w < WB)) + [WB]
    desc = np.argsort(-span, kind="stable")
    classes = []
    taken = 0
    for i in range(len(widths) - 1, -1, -1):
        lo = widths[i - 1] if i > 0 else -1
        if i > 0:
            need = int((span > lo).sum()) - taken
            n_w = max(0, -(-need // S) * S)
        else:
            n_w = n_tiles_pad - taken
        classes.append((widths[i], np.sort(desc[taken:taken + n_w])))
        taken += n_w
    classes = classes[::-1]

    n_pad = n_tiles_pad * TN
    cnt0 = np.bincount(idx0, minlength=n_pad).astype(np.float32)
    cnt1 = np.bincount(idx1, minlength=n_pad).astype(np.float32)
    r0 = (1.0 / np.maximum(cnt0, 1.0)).reshape(n_tiles_pad, TN)
    r1 = (1.0 / np.maximum(cnt1, 1.0)).reshape(n_tiles_pad, TN)

    class_data = []
    order_parts = []
    for w, tiles in classes:
        assert tiles.size % S == 0 and (span[tiles] <= w).all()
        ws = ws_all[tiles]
        gather = ws[:, None] + np.arange(w)[None, :]
        tbase = (tiles.astype(np.int64) * TN)[:, None]
        loc0 = (idx0_p[gather] - tbase).astype(np.int32)
        loc1 = (idx1_p[gather] - tbase).astype(np.int32)
        class_data.append(dict(W=w, n=tiles.size, ws=ws.astype(np.int32),
                               loc0=loc0, loc1=loc1,
                               r0=r0[tiles], r1=r1[tiles]))
        order_parts.append(tiles)
    order = np.concatenate(order_parts)
    assert order.size == n_tiles_pad
    inv_order = np.argsort(order).astype(np.int32)

    return dict(E=E, E_pad=E_pad, n_out=n_out, n_tiles_pad=n_tiles_pad,
                inv_order=inv_order, classes=class_data)


_P = _precompute()


def _body(W, ws_ref, loc0_ref, loc1_ref, r0_ref, r1_ref, xT_ref, o_ref):
    step = pl.program_id(0)
    for s in range(S):
        t = step * S + s
        start = pl.multiple_of(ws_ref[t], TN)
        xw = xT_ref[:, pl.ds(start, W)].astype(jnp.float32)
        iota = lax.broadcasted_iota(jnp.int32, (TN, W), 0)
        one = jnp.ones((), jnp.float32)
        zero = jnp.zeros((), jnp.float32)
        m0 = jnp.where(loc0_ref[s:s + 1, :] == iota, one, zero)
        m1 = jnp.where(loc1_ref[s:s + 1, :] == iota, one, zero)
        m = jnp.concatenate([m0, m1], axis=0)
        r = lax.dot_general(xw, m, (((1,), (1,)), ((), ())),
                            preferred_element_type=jnp.float32)
        o0 = r[:, :TN] * r0_ref[s:s + 1, :]
        o1 = r[:, TN:] * r1_ref[s:s + 1, :]
        rio = lax.broadcasted_iota(jnp.int32, (DPAD, TN), 0)
        mixed = jnp.where(rio < HDIM, (o1 - o0) * 0.5,
                          jnp.where(rio < 2 * HDIM, o1, o0))
        o_ref[:, s * TN:(s + 1) * TN] = mixed


def _scatter_call(W, n_class, ws, loc0, loc1, r0, r1, xT):
    steps = n_class // S
    return pl.pallas_call(
        functools.partial(_body, W),
        out_shape=jax.ShapeDtypeStruct((DPAD, n_class * TN), jnp.float32),
        grid_spec=pltpu.PrefetchScalarGridSpec(
            num_scalar_prefetch=1,
            grid=(steps,),
            in_specs=[
                pl.BlockSpec((S, W), lambda j, ws: (j, 0)),
                pl.BlockSpec((S, W), lambda j, ws: (j, 0)),
                pl.BlockSpec((S, TN), lambda j, ws: (j, 0)),
                pl.BlockSpec((S, TN), lambda j, ws: (j, 0)),
                pl.BlockSpec((DPAD, xT.shape[1]), lambda j, ws: (0, 0)),
            ],
            out_specs=pl.BlockSpec((DPAD, S * TN), lambda j, ws: (0, j)),
        ),
        compiler_params=pltpu.CompilerParams(
            dimension_semantics=("arbitrary",),
            vmem_limit_bytes=60 * 1024 * 1024),
        name=f"lg2graph_node_w{W}",
    )(jnp.asarray(ws), jnp.asarray(loc0), jnp.asarray(loc1),
      jnp.asarray(r0), jnp.asarray(r1), xT)


@jax.jit
def kernel(x):
    E_pad = _P["E_pad"]

    xT = jnp.pad(x.astype(jnp.bfloat16).T,
                 ((0, DPAD - DIM_INNER), (0, E_pad - _P["E"])))

    outs = [_scatter_call(c["W"], c["n"], c["ws"], c["loc0"], c["loc1"],
                          c["r0"], c["r1"], xT)
            for c in _P["classes"] if c["n"]]

    cat = jnp.concatenate(outs, axis=1) if len(outs) > 1 else outs[0]
    cat3 = cat.reshape(DPAD, _P["n_tiles_pad"], TN)
    full = jnp.take(cat3, jnp.asarray(_P["inv_order"]), axis=1)
    return full.reshape(DPAD, -1)[:DIM_INNER, :_P["n_out"]].T
